# Initial kernel scaffold; baseline (speedup 1.0000x reference)
import jax
import jax.numpy as jnp
from jax.experimental import pallas as pl


def kernel(input_ids, table, W1, b1, W2, b2):
    raise NotImplementedError("write your pallas kernel here")


# speedup vs baseline: 1.3455x; 1.3455x over previous
import functools

import jax
import jax.numpy as jnp
from jax import lax
from jax.experimental import pallas as pl
from jax.experimental.pallas import tpu as pltpu
from jax.experimental.pallas import tpu_sc as plsc

_D = 64
_B = 4096
_L = 200
_NUM_LABELS = 20
_MAX_NORM = 5.0

_NC = 2
_NS = 16
_NW = _NC * _NS
_SEQ_PER_W = _B // _NW
_CHUNK0 = 128
_CHUNK1 = _L - _CHUNK0

_DH = _D // 2
_NP = 128
_BLK = 512

_mesh = plsc.VectorSubcoreMesh(core_axis_name="c", subcore_axis_name="s")


@functools.partial(
    pl.kernel,
    out_type=jax.ShapeDtypeStruct((_B, _D), jnp.float32),
    mesh=_mesh,
    compiler_params=pltpu.CompilerParams(
        needs_layout_passes=False, use_tc_tiling_on_sc=False),
    scratch_types=[
        pltpu.VMEM((_SEQ_PER_W, _L), jnp.int32),
        pltpu.VMEM((2, _L, _D), jnp.float32),
        pltpu.VMEM((_SEQ_PER_W, _D), jnp.float32),
        pltpu.SemaphoreType.DMA,
        pltpu.SemaphoreType.DMA,
    ],
)
def _pooled_embed(ids_hbm, table_hbm, out_hbm, ids_v, rows_v, out_v, sem0, sem1):
    wid = lax.axis_index("c") * _NS + lax.axis_index("s")
    base = wid * _SEQ_PER_W
    pltpu.sync_copy(ids_hbm.at[pl.ds(base, _SEQ_PER_W)], ids_v)

    sems = (sem0, sem1)

    def _copies(s, b):
        sem = sems[b]
        c0 = pltpu.make_async_copy(
            table_hbm.at[ids_v.at[s, pl.ds(0, _CHUNK0)]],
            rows_v.at[b, pl.ds(0, _CHUNK0)], sem)
        c1 = pltpu.make_async_copy(
            table_hbm.at[ids_v.at[s, pl.ds(_CHUNK0, _CHUNK1)]],
            rows_v.at[b, pl.ds(_CHUNK0, _CHUNK1)], sem)
        return c0, c1

    def _issue(s, b):
        c0, c1 = _copies(s, b)
        c0.start()
        c1.start()

    def _wait(s, b):
        c0, c1 = _copies(s, b)
        c0.wait()
        c1.wait()

    def _compute(s, b):
        def row(r, acc):
            a0, a1, a2, a3 = acc
            v0 = rows_v[b, r, pl.ds(0, 16)]
            v1 = rows_v[b, r, pl.ds(16, 16)]
            v2 = rows_v[b, r, pl.ds(32, 16)]
            v3 = rows_v[b, r, pl.ds(48, 16)]
            p = v0 * v0 + v1 * v1 + v2 * v2 + v3 * v3
            nsq = jnp.sum(p)
            bits = lax.bitcast_convert_type(nsq, jnp.int32)
            y = lax.bitcast_convert_type(
                jnp.int32(0x5F3759DF) - (bits >> 1), jnp.float32)
            h = 0.5 * nsq
            y = y * (1.5 - h * y * y)
            y = y * (1.5 - h * y * y)
            scale = jnp.where(nsq > _MAX_NORM * _MAX_NORM, _MAX_NORM * y,
                              jnp.float32(1.0))
            sv = jnp.broadcast_to(scale, (16,))
            return (a0 + sv * v0, a1 + sv * v1, a2 + sv * v2, a3 + sv * v3)

        z = jnp.zeros((16,), jnp.float32)
        a0, a1, a2, a3 = lax.fori_loop(0, _L, row, (z, z, z, z), unroll=4)
        out_v[s, pl.ds(0, 16)] = a0
        out_v[s, pl.ds(16, 16)] = a1
        out_v[s, pl.ds(32, 16)] = a2
        out_v[s, pl.ds(48, 16)] = a3

    _issue(0, 0)

    def outer(i, carry):
        s0 = 2 * i
        _issue(s0 + 1, 1)
        _wait(s0, 0)
        _compute(s0, 0)

        @pl.when(s0 + 2 < _SEQ_PER_W)
        def _():
            _issue(s0 + 2, 0)

        _wait(s0 + 1, 1)
        _compute(s0 + 1, 1)
        return carry

    lax.fori_loop(0, _SEQ_PER_W // 2, outer, 0)
    pltpu.sync_copy(out_v, out_hbm.at[pl.ds(base, _SEQ_PER_W)])


def _mlp_body(x_ref, w1_ref, b1_ref, w2_ref, b2_ref, o_ref):
    x = jnp.maximum(x_ref[...], 0.0)
    y = jnp.dot(x, w1_ref[...], preferred_element_type=jnp.float32) + b1_ref[...]
    y = jnp.maximum(y, 0.0)
    o_ref[...] = jnp.dot(y, w2_ref[...], preferred_element_type=jnp.float32) + b2_ref[...]


_mlp = pl.pallas_call(
    _mlp_body,
    grid=(_B // _BLK,),
    in_specs=[
        pl.BlockSpec((_BLK, _D), lambda i: (i, 0)),
        pl.BlockSpec((_D, _NP), lambda i: (0, 0)),
        pl.BlockSpec((1, _NP), lambda i: (0, 0)),
        pl.BlockSpec((_NP, _NP), lambda i: (0, 0)),
        pl.BlockSpec((1, _NP), lambda i: (0, 0)),
    ],
    out_specs=pl.BlockSpec((_BLK, _NP), lambda i: (i, 0)),
    out_shape=jax.ShapeDtypeStruct((_B, _NP), jnp.float32),
)


def kernel(input_ids, table, W1, b1, W2, b2):
    ids = input_ids.astype(jnp.int32)
    sums = _pooled_embed(ids, table)
    w1p = jnp.zeros((_D, _NP), jnp.float32).at[:, :_DH].set(W1.T / float(_L))
    b1p = jnp.zeros((1, _NP), jnp.float32).at[0, :_DH].set(b1)
    w2p = jnp.zeros((_NP, _NP), jnp.float32).at[:_DH, :_NUM_LABELS].set(W2.T)
    b2p = jnp.zeros((1, _NP), jnp.float32).at[0, :_NUM_LABELS].set(b2)
    out = _mlp(sums, w1p, b1p, w2p, b2p)
    return out[:, :_NUM_LABELS]
